# Initial kernel scaffold; baseline (speedup 1.0000x reference)
import jax
import jax.numpy as jnp
from jax.experimental import pallas as pl


def kernel(x, rbf, sbf, idx_kj, idx_ji, W_rbf, W_sbf, Wkj, bkj, Wji, bji, Wb, rb0_W1, rb0_b1, rb0_W2, rb0_b2, Wlin, blin, ra0_W1, ra0_b1, ra0_W2, ra0_b2, ra1_W1, ra1_b1, ra1_W2, ra1_b2, Wout, bout):
    raise NotImplementedError("write your pallas kernel here")


# speedup vs baseline: 1.0955x; 1.0955x over previous
import functools

import jax
import jax.numpy as jnp
from jax.experimental import pallas as pl
from jax.experimental.pallas import tpu as pltpu

_E = 320000
_T = 640000
_H = 128
_BE = 4000


def _swish(v):
    return v * jax.nn.sigmoid(v)


def _dot(a, b):
    return jax.lax.dot_general(a, b, (((1,), (0,)), ((), ())),
                               preferred_element_type=jnp.float32)


def _mlp_body(h_ref, x_ref, w_ref, b_ref, o_ref):
    h = h_ref[...]
    x = x_ref[...]
    W = w_ref[...]
    B = b_ref[...]

    def lin(v, i):
        return _dot(v, W[i]) + B[i][None, :]

    h = h + _swish(lin(_swish(lin(h, 0)), 1))
    h = _swish(lin(h, 2)) + x
    h = h + _swish(lin(_swish(lin(h, 3)), 4))
    h = h + _swish(lin(_swish(lin(h, 5)), 6))
    o_ref[...] = _swish(lin(h, 7))


def _mlp_stack(h, x, Wstack, Bstack):
    grid = (_E // _BE,)
    return pl.pallas_call(
        _mlp_body,
        grid=grid,
        in_specs=[
            pl.BlockSpec((_BE, _H), lambda i: (i, 0)),
            pl.BlockSpec((_BE, _H), lambda i: (i, 0)),
            pl.BlockSpec((8, _H, _H), lambda i: (0, 0, 0)),
            pl.BlockSpec((8, _H), lambda i: (0, 0)),
        ],
        out_specs=pl.BlockSpec((_BE, _H), lambda i: (i, 0)),
        out_shape=jax.ShapeDtypeStruct((_E, _H), jnp.float32),
    )(h, x, Wstack, Bstack)


def kernel(x, rbf, sbf, idx_kj, idx_ji, W_rbf, W_sbf, Wkj, bkj, Wji, bji, Wb,
           rb0_W1, rb0_b1, rb0_W2, rb0_b2, Wlin, blin,
           ra0_W1, ra0_b1, ra0_W2, ra0_b2, ra1_W1, ra1_b1, ra1_W2, ra1_b2,
           Wout, bout):
    rbf_p = rbf @ W_rbf
    sbf_p = sbf @ W_sbf
    x_ji = _swish(x @ Wji + bji)
    x_kj = _swish(x @ Wkj + bkj) * rbf_p
    g = jnp.take(x_kj, idx_kj, axis=0)
    t = jnp.einsum('wj,wl,ijl->wi', sbf_p, g, Wb)
    agg = jax.ops.segment_sum(t, idx_ji, num_segments=_E)
    h = x_ji + agg

    Wstack = jnp.stack([rb0_W1, rb0_W2, Wlin, ra0_W1, ra0_W2,
                        ra1_W1, ra1_W2, Wout])
    Bstack = jnp.stack([rb0_b1, rb0_b2, blin, ra0_b1, ra0_b2,
                        ra1_b1, ra1_b2, bout])
    return _mlp_stack(h, x, Wstack, Bstack)
